# baseline (speedup 1.0000x reference)
import jax
import jax.numpy as jnp
from jax import lax
from jax.experimental import pallas as pl
from jax.experimental.pallas import tpu as pltpu
from jax.experimental.pallas import tpu_sc as plsc

BATCH = 4
SEQ = 8192
D = 1024
CHUNK = 32
NW = 32
ROWS_PER_W = SEQ // NW
NCHUNK = ROWS_PER_W // CHUNK


def _sc_body(x_hbm, tab_hbm, pos_hbm, out_hbm, pos_v, tab_buf, x_buf, sem):
    info = plsc.get_sparse_core_info()
    wid = lax.axis_index("s") * info.num_cores + lax.axis_index("c")
    off = wid * ROWS_PER_W

    def chunk_body(ci, carry):
        row0 = off + ci * CHUNK
        pltpu.sync_copy(pos_hbm.at[pl.ds(row0, CHUNK)], pos_v)
        pltpu.async_copy(tab_hbm.at[pos_v], tab_buf, sem).wait()

        def batch_body(b, carry2):
            xrow0 = b * SEQ + row0
            pltpu.sync_copy(x_hbm.at[pl.ds(xrow0, CHUNK)], x_buf)

            def r_body(r, carry3):
                def c_body(c, carry4):
                    sl = pl.ds(c * 16, 16)
                    x_buf[r, sl] = x_buf[r, sl] + tab_buf[r, sl]
                    return 0

                return lax.fori_loop(0, D // 16, c_body, 0)

            lax.fori_loop(0, CHUNK, r_body, 0)
            pltpu.sync_copy(x_buf, out_hbm.at[pl.ds(xrow0, CHUNK)])
            return 0

        lax.fori_loop(0, BATCH, batch_body, 0)
        return 0

    lax.fori_loop(0, NCHUNK, chunk_body, 0)


def kernel(x, pos_table, positions):
    xf = x.reshape(BATCH * SEQ, D)
    pos32 = positions.astype(jnp.int32)
    mesh = plsc.VectorSubcoreMesh(core_axis_name="c", subcore_axis_name="s")
    out = pl.kernel(
        _sc_body,
        out_type=jax.ShapeDtypeStruct((BATCH * SEQ, D), jnp.float32),
        mesh=mesh,
        scratch_types=[
            pltpu.VMEM((CHUNK,), jnp.int32),
            pltpu.VMEM((CHUNK, D), jnp.float32),
            pltpu.VMEM((CHUNK, D), jnp.float32),
            pltpu.SemaphoreType.DMA,
        ],
    )(xf, pos_table, pos32)
    return out.reshape(BATCH, SEQ, D)


# speedup vs baseline: 1.7693x; 1.7693x over previous
import jax
import jax.numpy as jnp
from jax import lax
from jax.experimental import pallas as pl
from jax.experimental.pallas import tpu as pltpu
from jax.experimental.pallas import tpu_sc as plsc

BATCH = 4
SEQ = 8192
D = 1024
CHUNK = 32
NW = 32
ROWS_PER_W = SEQ // NW
NCHUNK = ROWS_PER_W // CHUNK


def _sc_body(x_hbm, tab_hbm, pos_hbm, out_hbm, pos_v, tab_buf, x_buf, sem):
    info = plsc.get_sparse_core_info()
    wid = lax.axis_index("s") * info.num_cores + lax.axis_index("c")
    off = wid * ROWS_PER_W

    def chunk_body(ci, carry):
        row0 = off + ci * CHUNK
        pltpu.sync_copy(pos_hbm.at[pl.ds(row0, CHUNK)], pos_v)
        pltpu.async_copy(tab_hbm.at[pos_v], tab_buf, sem).wait()

        def batch_body(b, carry2):
            xrow0 = b * SEQ + row0
            pltpu.sync_copy(x_hbm.at[pl.ds(xrow0, CHUNK)], x_buf)

            @plsc.parallel_loop(0, CHUNK, 1, unroll=2)
            def r_body(r):
                for c in range(D // 16):
                    sl = pl.ds(c * 16, 16)
                    x_buf[r, sl] = x_buf[r, sl] + tab_buf[r, sl]
            pltpu.sync_copy(x_buf, out_hbm.at[pl.ds(xrow0, CHUNK)])
            return 0

        lax.fori_loop(0, BATCH, batch_body, 0)
        return 0

    lax.fori_loop(0, NCHUNK, chunk_body, 0)


def kernel(x, pos_table, positions):
    xf = x.reshape(BATCH * SEQ, D)
    pos32 = positions.astype(jnp.int32)
    mesh = plsc.VectorSubcoreMesh(core_axis_name="c", subcore_axis_name="s")
    out = pl.kernel(
        _sc_body,
        out_type=jax.ShapeDtypeStruct((BATCH * SEQ, D), jnp.float32),
        mesh=mesh,
        scratch_types=[
            pltpu.VMEM((CHUNK,), jnp.int32),
            pltpu.VMEM((CHUNK, D), jnp.float32),
            pltpu.VMEM((CHUNK, D), jnp.float32),
            pltpu.SemaphoreType.DMA,
        ],
    )(xf, pos_table, pos32)
    return out.reshape(BATCH, SEQ, D)


# speedup vs baseline: 3.1591x; 1.7855x over previous
import jax
import jax.numpy as jnp
from jax import lax
from jax.experimental import pallas as pl
from jax.experimental.pallas import tpu as pltpu
from jax.experimental.pallas import tpu_sc as plsc

BATCH = 4
SEQ = 8192
D = 1024
CHUNK = 32
NW = 32
ROWS_PER_W = SEQ // NW
NCHUNK = ROWS_PER_W // CHUNK

SEQ_BLOCK = 2048


def _sc_gather_body(tab_hbm, pos_hbm, out_hbm, pos_v, tab_buf, sem):
    info = plsc.get_sparse_core_info()
    wid = lax.axis_index("s") * info.num_cores + lax.axis_index("c")
    off = wid * ROWS_PER_W

    def chunk_body(ci, carry):
        row0 = off + ci * CHUNK
        pltpu.sync_copy(pos_hbm.at[pl.ds(row0, CHUNK)], pos_v)
        pltpu.async_copy(tab_hbm.at[pos_v], tab_buf, sem).wait()
        pltpu.sync_copy(tab_buf, out_hbm.at[pl.ds(row0, CHUNK)])
        return 0

    lax.fori_loop(0, NCHUNK, chunk_body, 0)


def _sc_gather(pos_table, pos32):
    mesh = plsc.VectorSubcoreMesh(core_axis_name="c", subcore_axis_name="s")
    return pl.kernel(
        _sc_gather_body,
        out_type=jax.ShapeDtypeStruct((SEQ, D), jnp.float32),
        mesh=mesh,
        scratch_types=[
            pltpu.VMEM((CHUNK,), jnp.int32),
            pltpu.VMEM((CHUNK, D), jnp.float32),
            pltpu.SemaphoreType.DMA,
        ],
    )(pos_table, pos32)


def _tc_add_kernel(x_ref, emb_ref, out_ref):
    out_ref[...] = x_ref[...] + emb_ref[...]


def _tc_add(x, pos_emb):
    batch, max_len, d_model = x.shape
    ns = max_len // SEQ_BLOCK
    return pl.pallas_call(
        _tc_add_kernel,
        grid=(ns, batch),
        in_specs=[
            pl.BlockSpec((1, SEQ_BLOCK, d_model), lambda s, b: (b, s, 0)),
            pl.BlockSpec((SEQ_BLOCK, d_model), lambda s, b: (s, 0)),
        ],
        out_specs=pl.BlockSpec((1, SEQ_BLOCK, d_model), lambda s, b: (b, s, 0)),
        out_shape=jax.ShapeDtypeStruct(x.shape, x.dtype),
        compiler_params=pltpu.CompilerParams(
            dimension_semantics=("arbitrary", "arbitrary"),
        ),
    )(x, pos_emb)


def kernel(x, pos_table, positions):
    pos32 = positions.astype(jnp.int32)
    pos_emb = _sc_gather(pos_table, pos32)
    return _tc_add(x, pos_emb)


# speedup vs baseline: 3.2994x; 1.0444x over previous
import jax
import jax.numpy as jnp
from jax import lax
from jax.experimental import pallas as pl
from jax.experimental.pallas import tpu as pltpu
from jax.experimental.pallas import tpu_sc as plsc

BATCH = 4
SEQ = 8192
D = 1024
CHUNK = 32
NW = 32
ROWS_PER_W = SEQ // NW
NCHUNK = ROWS_PER_W // CHUNK

SEQ_BLOCK = 2048


def _sc_gather_body(tab_hbm, pos_hbm, out_hbm, pos_a, pos_b, buf_a, buf_b,
                    gsem_a, gsem_b, ssem_a, ssem_b):
    info = plsc.get_sparse_core_info()
    wid = lax.axis_index("s") * info.num_cores + lax.axis_index("c")
    off = wid * ROWS_PER_W

    pos_bufs = (pos_a, pos_b)
    bufs = (buf_a, buf_b)
    gsems = (gsem_a, gsem_b)
    ssems = (ssem_a, ssem_b)

    gathers = [None] * NCHUNK
    stores = [None] * NCHUNK

    def start_gather(ci):
        p = ci % 2
        row0 = off + ci * CHUNK
        pltpu.sync_copy(pos_hbm.at[pl.ds(row0, CHUNK)], pos_bufs[p])
        gathers[ci] = pltpu.async_copy(tab_hbm.at[pos_bufs[p]], bufs[p], gsems[p])

    start_gather(0)
    for ci in range(NCHUNK):
        p = ci % 2
        if ci + 1 < NCHUNK:
            if stores[ci - 1] is not None:
                stores[ci - 1].wait()
            start_gather(ci + 1)
        gathers[ci].wait()
        row0 = off + ci * CHUNK
        stores[ci] = pltpu.async_copy(bufs[p], out_hbm.at[pl.ds(row0, CHUNK)],
                                      ssems[p])
    stores[NCHUNK - 2].wait()
    stores[NCHUNK - 1].wait()


def _sc_gather(pos_table, pos32):
    mesh = plsc.VectorSubcoreMesh(core_axis_name="c", subcore_axis_name="s")
    return pl.kernel(
        _sc_gather_body,
        out_type=jax.ShapeDtypeStruct((SEQ, D), jnp.float32),
        mesh=mesh,
        scratch_types=[
            pltpu.VMEM((CHUNK,), jnp.int32),
            pltpu.VMEM((CHUNK,), jnp.int32),
            pltpu.VMEM((CHUNK, D), jnp.float32),
            pltpu.VMEM((CHUNK, D), jnp.float32),
            pltpu.SemaphoreType.DMA,
            pltpu.SemaphoreType.DMA,
            pltpu.SemaphoreType.DMA,
            pltpu.SemaphoreType.DMA,
        ],
    )(pos_table, pos32)


def _tc_add_kernel(x_ref, emb_ref, out_ref):
    out_ref[...] = x_ref[...] + emb_ref[...]


def _tc_add(x, pos_emb):
    batch, max_len, d_model = x.shape
    ns = max_len // SEQ_BLOCK
    return pl.pallas_call(
        _tc_add_kernel,
        grid=(ns, batch),
        in_specs=[
            pl.BlockSpec((1, SEQ_BLOCK, d_model), lambda s, b: (b, s, 0)),
            pl.BlockSpec((SEQ_BLOCK, d_model), lambda s, b: (s, 0)),
        ],
        out_specs=pl.BlockSpec((1, SEQ_BLOCK, d_model), lambda s, b: (b, s, 0)),
        out_shape=jax.ShapeDtypeStruct(x.shape, x.dtype),
        compiler_params=pltpu.CompilerParams(
            dimension_semantics=("arbitrary", "arbitrary"),
        ),
    )(x, pos_emb)


def kernel(x, pos_table, positions):
    pos32 = positions.astype(jnp.int32)
    pos_emb = _sc_gather(pos_table, pos32)
    return _tc_add(x, pos_emb)
